# baseline (speedup 1.0000x reference)
import functools

import jax
import jax.numpy as jnp
import numpy as np
from jax.experimental import pallas as pl
from jax.experimental.pallas import tpu as pltpu
from jax.experimental.pallas import tpu_sc as plsc

_TOP_K = 16


def _bdot(a, b):
    return jax.lax.dot_general(a.astype(jnp.bfloat16), b.astype(jnp.bfloat16),
                               (((1,), (1,)), ((), ())),
                               preferred_element_type=jnp.float32)


def _kv_body(side_ref, rel_ref, kw_ref, kb_ref, vw_ref, vb_ref, k_out, v_out):
    side = side_ref[...]
    kin = side * rel_ref[...]
    k_out[...] = _bdot(kin, kw_ref[...]) + kb_ref[...]
    v_out[...] = _bdot(side, vw_ref[...]) + vb_ref[...]


def _topk_vals_cols(s, n_side):
    p = 128
    nfull = n_side // p
    tail_w = n_side - nfull * p
    r = s.shape[0]
    neg = jnp.float32(-1e30)
    sl = [s[:, i * p:(i + 1) * p] for i in range(nfull)]
    cms = []
    for lv in range(4):
        cm = neg * jnp.ones_like(sl[0])
        for x in sl:
            hit = jnp.zeros_like(x, jnp.bool_)
            for prev in cms:
                hit = hit | (x == prev)
            cm = jnp.maximum(cm, jnp.where(hit, neg, x))
        cms.append(cm)
    idxs = []
    for lv in range(4):
        idx = jnp.zeros(cms[0].shape, jnp.int32)
        for i in range(nfull - 1, -1, -1):
            idx = jnp.where(sl[i] == cms[lv], jnp.int32(i), idx)
        idxs.append(idx)
    lane = jax.lax.broadcasted_iota(jnp.int32, cms[0].shape, 1)
    cols = [ix * p + lane for ix in idxs]
    if tail_w:
        tail = s[:, nfull * p:]
        negt = neg * jnp.ones_like(tail)
        tlane = jax.lax.broadcasted_iota(jnp.int32, tail.shape, 1) + nfull * p
        w = jnp.concatenate([cms[0], tail], axis=1)
        colw = jnp.concatenate([cols[0], tlane], axis=1)
        nxtv = [jnp.concatenate([cms[lv], negt], axis=1) for lv in (1, 2, 3)]
        nxtc = [jnp.concatenate([cols[lv], jnp.zeros_like(tlane)], axis=1)
                for lv in (1, 2, 3)]
    else:
        w, colw = cms[0], cols[0]
        nxtv = cms[1:]
        nxtc = cols[1:]
    lvl = jnp.zeros(w.shape, jnp.int32)
    big = jnp.int32(2 ** 30)
    tvals, tcols = [], []
    for i in range(_TOP_K):
        t = jnp.max(w, axis=1, keepdims=True)
        sel = w == t
        c = jnp.min(jnp.where(sel, colw, big), axis=1, keepdims=True)
        tvals.append(t)
        tcols.append(c)
        if i < _TOP_K - 1:
            lvl = lvl + sel.astype(jnp.int32)
            nv = jnp.where(lvl == 1, nxtv[0],
                           jnp.where(lvl == 2, nxtv[1],
                                     jnp.where(lvl == 3, nxtv[2], neg)))
            ncol = jnp.where(lvl == 1, nxtc[0],
                             jnp.where(lvl == 2, nxtc[1],
                                       jnp.where(lvl == 3, nxtc[2], 0)))
            w = jnp.where(sel, nv, w)
            colw = jnp.where(sel, ncol, colw)
    return jnp.concatenate(tvals, axis=1), jnp.concatenate(tcols, axis=1)


def _main_body(ego_ref, qw_ref, qb_ref, k_ref, w_out, c_out, *, scale, n_side):
    q = _bdot(ego_ref[...], qw_ref[...]) + qb_ref[...]
    s = _bdot(q, k_ref[...]) / scale
    tvals, tcols = _topk_vals_cols(s, n_side)
    e = jnp.exp(tvals - tvals[:, 0:1])
    w_out[...] = e / jnp.sum(e, axis=1, keepdims=True)
    c_out[...] = tcols


def _build_tc(n_ego, n_side, d, r_block, kv_block):
    scale = np.float32(np.sqrt(d))
    kv = pl.pallas_call(
        _kv_body,
        grid=(n_side // kv_block,),
        in_specs=[
            pl.BlockSpec((kv_block, d), lambda i: (i, 0)),
            pl.BlockSpec((kv_block, d), lambda i: (i, 0)),
            pl.BlockSpec((d, d), lambda i: (0, 0)),
            pl.BlockSpec((1, d), lambda i: (0, 0)),
            pl.BlockSpec((d, d), lambda i: (0, 0)),
            pl.BlockSpec((1, d), lambda i: (0, 0)),
        ],
        out_specs=[
            pl.BlockSpec((kv_block, d), lambda i: (i, 0)),
            pl.BlockSpec((kv_block, d), lambda i: (i, 0)),
        ],
        out_shape=[
            jax.ShapeDtypeStruct((n_side, d), jnp.float32),
            jax.ShapeDtypeStruct((n_side, d), jnp.float32),
        ],
    )
    main = pl.pallas_call(
        functools.partial(_main_body, scale=scale, n_side=n_side),
        grid=(n_ego // r_block,),
        in_specs=[
            pl.BlockSpec((r_block, d), lambda i: (i, 0)),
            pl.BlockSpec((d, d), lambda i: (0, 0)),
            pl.BlockSpec((1, d), lambda i: (0, 0)),
            pl.BlockSpec((n_side, d), lambda i: (0, 0)),
        ],
        out_specs=[
            pl.BlockSpec((r_block, _TOP_K), lambda i: (i, 0)),
            pl.BlockSpec((r_block, _TOP_K), lambda i: (i, 0)),
        ],
        out_shape=[
            jax.ShapeDtypeStruct((n_ego, _TOP_K), jnp.float32),
            jax.ShapeDtypeStruct((n_ego, _TOP_K), jnp.int32),
        ],
    )
    return kv, main


def _build_sc(n_pad, d, rows_per_tile, batch):
    mesh = plsc.VectorSubcoreMesh(core_axis_name="c", subcore_axis_name="s")
    nb = rows_per_tile // batch

    @functools.partial(
        pl.kernel, mesh=mesh,
        out_type=jax.ShapeDtypeStruct((n_pad, d), jnp.float32),
        scratch_types=[
            pltpu.VMEM((batch * _TOP_K,), jnp.int32),
            pltpu.VMEM((batch, _TOP_K), jnp.float32),
            pltpu.VMEM((batch * _TOP_K, d), jnp.float32),
            pltpu.VMEM((batch, d), jnp.float32),
            pltpu.SemaphoreType.DMA,
        ])
    def sck(v_hbm, idxf_hbm, w_hbm, out_hbm, idx_v, w_v, rows_v, acc_v, sem):
        nc = 2
        wid = jax.lax.axis_index("s") * nc + jax.lax.axis_index("c")
        base0 = wid * rows_per_tile

        def body(b, _):
            base = base0 + b * batch
            pltpu.sync_copy(idxf_hbm.at[pl.ds(base * _TOP_K, batch * _TOP_K)],
                            idx_v)
            pltpu.async_copy(v_hbm.at[idx_v], rows_v, sem).wait()
            pltpu.sync_copy(w_hbm.at[pl.ds(base, batch), :], w_v)
            for r in range(batch):
                wrow = w_v[r, :]
                for c in range(d // 16):
                    acc = rows_v[r * _TOP_K, pl.ds(c * 16, 16)] * wrow[0]
                    for j in range(1, _TOP_K):
                        acc = acc + (rows_v[r * _TOP_K + j, pl.ds(c * 16, 16)]
                                     * wrow[j])
                    acc_v[r, pl.ds(c * 16, 16)] = acc
            pltpu.sync_copy(acc_v, out_hbm.at[pl.ds(base, batch), :])
            return _

        jax.lax.fori_loop(0, nb, body, 0)

    return sck


def kernel(ego_emb, side_emb, rel_emb, q_w, q_b, k_w, k_b, v_w, v_b):
    n_ego, d = ego_emb.shape
    n_side = side_emb.shape[0]
    r_block = 400 if n_ego % 400 == 0 else n_ego
    kv_block = 2000 if n_side % 2000 == 0 else n_side
    kv, main = _build_tc(n_ego, n_side, d, r_block, kv_block)
    k_mat, v_mat = kv(side_emb, rel_emb, k_w, k_b.reshape(1, d),
                      v_w, v_b.reshape(1, d))
    w16, c16 = main(ego_emb, q_w, q_b.reshape(1, d), k_mat)

    n_tiles = 32
    batch = 8
    rows_per_tile = -(-n_ego // (n_tiles * batch)) * batch
    n_pad = rows_per_tile * n_tiles
    pad = n_pad - n_ego
    w_p = jnp.pad(w16, ((0, pad), (0, 0)))
    c_p = jnp.pad(c16, ((0, pad), (0, 0)))
    sck = _build_sc(n_pad, d, rows_per_tile, batch)
    out = sck(v_mat, c_p.reshape(-1), w_p)
    return out[:n_ego]


# speedup vs baseline: 1.0915x; 1.0915x over previous
import functools

import jax
import jax.numpy as jnp
import numpy as np
from jax.experimental import pallas as pl
from jax.experimental.pallas import tpu as pltpu
from jax.experimental.pallas import tpu_sc as plsc

_TOP_K = 16


def _bdot(a, b):
    return jax.lax.dot_general(a.astype(jnp.bfloat16), b.astype(jnp.bfloat16),
                               (((1,), (1,)), ((), ())),
                               preferred_element_type=jnp.float32)


def _kv_body(side_ref, rel_ref, kw_ref, kb_ref, vw_ref, vb_ref, k_out, v_out):
    side = side_ref[...]
    kin = side * rel_ref[...]
    k_out[...] = _bdot(kin, kw_ref[...]) + kb_ref[...]
    v_out[...] = _bdot(side, vw_ref[...]) + vb_ref[...]


def _topk_vals_cols(s, n_side):
    p = 128
    nfull = n_side // p
    tail_w = n_side - nfull * p
    neg = jnp.float32(-1e30)
    half = nfull // 2
    groups = [list(range(0, half)), list(range(half, nfull))]

    def peel3(slice_ids):
        sl = [(i, s[:, i * p:(i + 1) * p]) for i in slice_ids]
        cm1 = sl[0][1]
        for _, x in sl[1:]:
            cm1 = jnp.maximum(cm1, x)
        cms = [cm1]
        idxs = []
        for lv in (1, 2):
            cmn = neg * jnp.ones_like(cm1)
            idx = jnp.zeros(cm1.shape, jnp.int32)
            for i, x in sl:
                eq_last = x == cms[-1]
                hit = eq_last
                for prev in cms[:-1]:
                    hit = hit | (x == prev)
                idx = jnp.where(eq_last, jnp.int32(i), idx)
                cmn = jnp.maximum(cmn, jnp.where(hit, neg, x))
            idxs.append(idx)
            cms.append(cmn)
        idx3 = jnp.zeros(cm1.shape, jnp.int32)
        for i, x in sl:
            idx3 = jnp.where(x == cms[-1], jnp.int32(i), idx3)
        idxs.append(idx3)
        lane = jax.lax.broadcasted_iota(jnp.int32, cm1.shape, 1)
        return cms, [ix * p + lane for ix in idxs]

    cms_a, cols_a = peel3(groups[0])
    cms_b, cols_b = peel3(groups[1])
    parts_v = [[cms_a[lv], cms_b[lv]] for lv in range(3)]
    parts_c = [[cols_a[lv], cols_b[lv]] for lv in range(3)]
    if tail_w:
        tail = s[:, nfull * p:]
        negt = neg * jnp.ones_like(tail)
        tlane = jax.lax.broadcasted_iota(jnp.int32, tail.shape, 1) + nfull * p
        parts_v[0].append(tail)
        parts_c[0].append(tlane)
        for lv in (1, 2):
            parts_v[lv].append(negt)
            parts_c[lv].append(jnp.zeros_like(tlane))
    w = jnp.concatenate(parts_v[0], axis=1)
    colw = jnp.concatenate(parts_c[0], axis=1)
    nxtv = [jnp.concatenate(parts_v[lv], axis=1) for lv in (1, 2)]
    nxtc = [jnp.concatenate(parts_c[lv], axis=1) for lv in (1, 2)]
    lvl = jnp.zeros(w.shape, jnp.int32)
    big = jnp.int32(2 ** 30)
    tvals, tcols = [], []
    for i in range(_TOP_K):
        t = jnp.max(w, axis=1, keepdims=True)
        sel = w == t
        c = jnp.min(jnp.where(sel, colw, big), axis=1, keepdims=True)
        tvals.append(t)
        tcols.append(c)
        if i < _TOP_K - 1:
            lvl = lvl + sel.astype(jnp.int32)
            nv = jnp.where(lvl == 1, nxtv[0],
                           jnp.where(lvl == 2, nxtv[1], neg))
            ncol = jnp.where(lvl == 1, nxtc[0],
                             jnp.where(lvl == 2, nxtc[1], 0))
            w = jnp.where(sel, nv, w)
            colw = jnp.where(sel, ncol, colw)
    return jnp.concatenate(tvals, axis=1), jnp.concatenate(tcols, axis=1)


def _main_body(ego_ref, qw_ref, qb_ref, k_ref, w_out, c_out, *, scale, n_side):
    q = _bdot(ego_ref[...], qw_ref[...]) + qb_ref[...]
    s = _bdot(q, k_ref[...]) / scale
    tvals, tcols = _topk_vals_cols(s, n_side)
    e = jnp.exp(tvals - tvals[:, 0:1])
    w_out[...] = e / jnp.sum(e, axis=1, keepdims=True)
    c_out[...] = tcols


def _build_tc(n_ego, n_side, d, r_block, kv_block):
    scale = np.float32(np.sqrt(d))
    kv = pl.pallas_call(
        _kv_body,
        grid=(n_side // kv_block,),
        in_specs=[
            pl.BlockSpec((kv_block, d), lambda i: (i, 0)),
            pl.BlockSpec((kv_block, d), lambda i: (i, 0)),
            pl.BlockSpec((d, d), lambda i: (0, 0)),
            pl.BlockSpec((1, d), lambda i: (0, 0)),
            pl.BlockSpec((d, d), lambda i: (0, 0)),
            pl.BlockSpec((1, d), lambda i: (0, 0)),
        ],
        out_specs=[
            pl.BlockSpec((kv_block, d), lambda i: (i, 0)),
            pl.BlockSpec((kv_block, d), lambda i: (i, 0)),
        ],
        out_shape=[
            jax.ShapeDtypeStruct((n_side, d), jnp.float32),
            jax.ShapeDtypeStruct((n_side, d), jnp.float32),
        ],
    )
    main = pl.pallas_call(
        functools.partial(_main_body, scale=scale, n_side=n_side),
        grid=(n_ego // r_block,),
        in_specs=[
            pl.BlockSpec((r_block, d), lambda i: (i, 0)),
            pl.BlockSpec((d, d), lambda i: (0, 0)),
            pl.BlockSpec((1, d), lambda i: (0, 0)),
            pl.BlockSpec((n_side, d), lambda i: (0, 0)),
        ],
        out_specs=[
            pl.BlockSpec((r_block, _TOP_K), lambda i: (i, 0)),
            pl.BlockSpec((r_block, _TOP_K), lambda i: (i, 0)),
        ],
        out_shape=[
            jax.ShapeDtypeStruct((n_ego, _TOP_K), jnp.float32),
            jax.ShapeDtypeStruct((n_ego, _TOP_K), jnp.int32),
        ],
    )
    return kv, main


def _build_sc(n_pad, d, rows_per_tile, batch):
    mesh = plsc.VectorSubcoreMesh(core_axis_name="c", subcore_axis_name="s")
    nb = rows_per_tile // batch

    @functools.partial(
        pl.kernel, mesh=mesh,
        out_type=jax.ShapeDtypeStruct((n_pad, d), jnp.float32),
        scratch_types=[
            pltpu.VMEM((batch * _TOP_K,), jnp.int32),
            pltpu.VMEM((batch, _TOP_K), jnp.float32),
            pltpu.VMEM((batch * _TOP_K, d), jnp.float32),
            pltpu.VMEM((batch, d), jnp.float32),
            pltpu.SemaphoreType.DMA,
        ])
    def sck(v_hbm, idxf_hbm, w_hbm, out_hbm, idx_v, w_v, rows_v, acc_v, sem):
        nc = 2
        wid = jax.lax.axis_index("s") * nc + jax.lax.axis_index("c")
        base0 = wid * rows_per_tile

        def body(b, _):
            base = base0 + b * batch
            pltpu.sync_copy(idxf_hbm.at[pl.ds(base * _TOP_K, batch * _TOP_K)],
                            idx_v)
            pltpu.async_copy(v_hbm.at[idx_v], rows_v, sem).wait()
            pltpu.sync_copy(w_hbm.at[pl.ds(base, batch), :], w_v)
            for r in range(batch):
                wrow = w_v[r, :]
                for c in range(d // 16):
                    acc = rows_v[r * _TOP_K, pl.ds(c * 16, 16)] * wrow[0]
                    for j in range(1, _TOP_K):
                        acc = acc + (rows_v[r * _TOP_K + j, pl.ds(c * 16, 16)]
                                     * wrow[j])
                    acc_v[r, pl.ds(c * 16, 16)] = acc
            pltpu.sync_copy(acc_v, out_hbm.at[pl.ds(base, batch), :])
            return _

        jax.lax.fori_loop(0, nb, body, 0)

    return sck


def kernel(ego_emb, side_emb, rel_emb, q_w, q_b, k_w, k_b, v_w, v_b):
    n_ego, d = ego_emb.shape
    n_side = side_emb.shape[0]
    r_block = 400 if n_ego % 400 == 0 else n_ego
    kv_block = 2000 if n_side % 2000 == 0 else n_side
    kv, main = _build_tc(n_ego, n_side, d, r_block, kv_block)
    k_mat, v_mat = kv(side_emb, rel_emb, k_w, k_b.reshape(1, d),
                      v_w, v_b.reshape(1, d))
    w16, c16 = main(ego_emb, q_w, q_b.reshape(1, d), k_mat)

    n_tiles = 32
    batch = 8
    rows_per_tile = -(-n_ego // (n_tiles * batch)) * batch
    n_pad = rows_per_tile * n_tiles
    pad = n_pad - n_ego
    w_p = jnp.pad(w16, ((0, pad), (0, 0)))
    c_p = jnp.pad(c16, ((0, pad), (0, 0)))
    sck = _build_sc(n_pad, d, rows_per_tile, batch)
    out = sck(v_mat, c_p.reshape(-1), w_p)
    return out[:n_ego]
